# Initial kernel scaffold; baseline (speedup 1.0000x reference)
import jax
import jax.numpy as jnp
from jax.experimental import pallas as pl


def kernel(species_idx, move_indices, ability_idx, item_idx, type_indices, move_type_indices, float_features, pokemon_table, move_table, ability_table, item_table, type_table, W1, b1, W2, b2):
    raise NotImplementedError("write your pallas kernel here")


# speedup vs baseline: 2.6996x; 2.6996x over previous
import functools

import jax
import jax.numpy as jnp
from jax import lax
from jax.experimental import pallas as pl
from jax.experimental.pallas import tpu as pltpu
from jax.experimental.pallas import tpu_sc as plsc

_B = 16384
_NW = 32
_ROWS_PER_W = _B // _NW
_CHUNK = 128
_NCHUNK = _ROWS_PER_W // _CHUNK


def _sc_gather_kernel(
    sp_idx_h, mv0_h, mv1_h, mv2_h, mv3_h, ab_idx_h, it_idx_h,
    t1_idx_h, t2_idx_h, mt0_h, mt1_h, mt2_h, mt3_h,
    pok_tab, mv_tab, ab_tab, it_tab, ty_tab,
    se_o, m0_o, m1_o, m2_o, m3_o, ae_o, ie_o, t1_o, t2_o,
    mt0_o, mt1_o, mt2_o, mt3_o, rm_o, rt_o,
    sp_i, mv0_i, mv1_i, mv2_i, mv3_i, ab_i, it_i, t1_i, t2_i,
    mt0_i, mt1_i, mt2_i, mt3_i,
    se_r, m0_r, m1_r, m2_r, m3_r, ae_r, ie_r, t1_r, t2_r,
    mt0_r, mt1_r, mt2_r, mt3_r,
    rm_v, rt_v, sem,
):
  wid = lax.axis_index("c") * 16 + lax.axis_index("s")
  base = wid * _ROWS_PER_W

  mv_is = (mv0_i, mv1_i, mv2_i, mv3_i)
  mt_is = (mt0_i, mt1_i, mt2_i, mt3_i)
  mv_hs = (mv0_h, mv1_h, mv2_h, mv3_h)
  mt_hs = (mt0_h, mt1_h, mt2_h, mt3_h)
  mv_rs = (m0_r, m1_r, m2_r, m3_r)
  mt_rs = (mt0_r, mt1_r, mt2_r, mt3_r)
  mv_os = (m0_o, m1_o, m2_o, m3_o)
  mt_os = (mt0_o, mt1_o, mt2_o, mt3_o)

  for c in range(_NCHUNK):
    sl = pl.ds(base + c * _CHUNK, _CHUNK)

    pltpu.sync_copy(sp_idx_h.at[sl], sp_i)
    pltpu.sync_copy(ab_idx_h.at[sl], ab_i)
    pltpu.sync_copy(it_idx_h.at[sl], it_i)
    pltpu.sync_copy(t1_idx_h.at[sl], t1_i)
    pltpu.sync_copy(t2_idx_h.at[sl], t2_i)
    for j in range(4):
      pltpu.sync_copy(mv_hs[j].at[sl], mv_is[j])
      pltpu.sync_copy(mt_hs[j].at[sl], mt_is[j])

    for i in range(_CHUNK // 16):
      s2 = pl.ds(i * 16, 16)
      cm = jnp.zeros((16,), jnp.float32)
      ct = jnp.zeros((16,), jnp.float32)
      for j in range(4):
        v = mv_is[j][s2]
        nz = v != 0
        cm = cm + jnp.where(nz, 1.0, 0.0)
        mv_is[j][s2] = jnp.where(nz, v, 920)
        w = mt_is[j][s2]
        nzt = w != 0
        ct = ct + jnp.where(nzt, 1.0, 0.0)
        mt_is[j][s2] = jnp.where(nzt, w, 19)
      rm_v[s2] = 1.0 / jnp.maximum(cm, 1.0)
      rt_v[s2] = 1.0 / jnp.maximum(ct, 1.0)

    handles = [
        pltpu.async_copy(pok_tab.at[sp_i], se_r, sem),
        pltpu.async_copy(ab_tab.at[ab_i], ae_r, sem),
        pltpu.async_copy(it_tab.at[it_i], ie_r, sem),
        pltpu.async_copy(ty_tab.at[t1_i], t1_r, sem),
        pltpu.async_copy(ty_tab.at[t2_i], t2_r, sem),
    ]
    for j in range(4):
      handles.append(pltpu.async_copy(mv_tab.at[mv_is[j]], mv_rs[j], sem))
      handles.append(pltpu.async_copy(ty_tab.at[mt_is[j]], mt_rs[j], sem))
    for h in handles:
      h.wait()

    pltpu.sync_copy(se_r, se_o.at[sl])
    pltpu.sync_copy(ae_r, ae_o.at[sl])
    pltpu.sync_copy(ie_r, ie_o.at[sl])
    pltpu.sync_copy(t1_r, t1_o.at[sl])
    pltpu.sync_copy(t2_r, t2_o.at[sl])
    for j in range(4):
      pltpu.sync_copy(mv_rs[j], mv_os[j].at[sl])
      pltpu.sync_copy(mt_rs[j], mt_os[j].at[sl])
    pltpu.sync_copy(rm_v, rm_o.at[sl])
    pltpu.sync_copy(rt_v, rt_o.at[sl])


def _make_sc_gather():
  f32 = jnp.float32
  i32 = jnp.int32
  out_type = [
      jax.ShapeDtypeStruct((_B, 48), f32),
      *[jax.ShapeDtypeStruct((_B, 32), f32) for _ in range(4)],
      jax.ShapeDtypeStruct((_B, 16), f32),
      jax.ShapeDtypeStruct((_B, 16), f32),
      jax.ShapeDtypeStruct((_B, 16), f32),
      jax.ShapeDtypeStruct((_B, 16), f32),
      *[jax.ShapeDtypeStruct((_B, 16), f32) for _ in range(4)],
      jax.ShapeDtypeStruct((_B,), f32),
      jax.ShapeDtypeStruct((_B,), f32),
  ]
  scratch = [
      *[pltpu.VMEM((_CHUNK,), i32) for _ in range(13)],
      pltpu.VMEM((_CHUNK, 48), f32),
      *[pltpu.VMEM((_CHUNK, 32), f32) for _ in range(4)],
      *[pltpu.VMEM((_CHUNK, 16), f32) for _ in range(8)],
      pltpu.VMEM((_CHUNK,), f32),
      pltpu.VMEM((_CHUNK,), f32),
      pltpu.SemaphoreType.DMA,
  ]
  mesh = plsc.VectorSubcoreMesh(core_axis_name="c", subcore_axis_name="s")
  return pl.kernel(
      _sc_gather_kernel, out_type=out_type, mesh=mesh,
      scratch_types=scratch,
      compiler_params=pltpu.CompilerParams(use_tc_tiling_on_sc=False))


_sc_gather = _make_sc_gather()

_BS = 512


def _tc_mlp_kernel(se, m0, m1, m2, m3, ae, ie, t1, t2,
                   mt0, mt1, mt2, mt3, rm, rt, ff, w1, b1, w2, b2, out):
  msum = (m0[...] + m1[...] + m2[...] + m3[...]) * rm[...]
  mtsum = (mt0[...] + mt1[...] + mt2[...] + mt3[...]) * rt[...]
  x = jnp.concatenate(
      [se[...], msum, ae[...], ie[...], t1[...], t2[...], mtsum, ff[...],
       jnp.zeros((_BS, 1), jnp.float32)], axis=-1)
  h = jnp.maximum(
      jnp.dot(x, w1[...], preferred_element_type=jnp.float32) + b1[...], 0.0)
  out[...] = jnp.maximum(
      jnp.dot(h, w2[...], preferred_element_type=jnp.float32) + b2[...], 0.0)


def _make_tc_mlp():
  grid = (_B // _BS,)
  def bspec(cols):
    return pl.BlockSpec((_BS, cols), lambda i: (i, 0))
  in_specs = [
      bspec(48),
      *[bspec(32) for _ in range(4)],
      *[bspec(16) for _ in range(4)],
      *[bspec(16) for _ in range(4)],
      bspec(1), bspec(1),
      bspec(31),
      pl.BlockSpec((192, 256), lambda i: (0, 0)),
      pl.BlockSpec((1, 256), lambda i: (0, 0)),
      pl.BlockSpec((256, 128), lambda i: (0, 0)),
      pl.BlockSpec((1, 128), lambda i: (0, 0)),
  ]
  return pl.pallas_call(
      _tc_mlp_kernel,
      grid=grid,
      in_specs=in_specs,
      out_specs=pl.BlockSpec((_BS, 128), lambda i: (i, 0)),
      out_shape=jax.ShapeDtypeStruct((_B, 128), jnp.float32),
  )


_tc_mlp = _make_tc_mlp()


def kernel(species_idx, move_indices, ability_idx, item_idx, type_indices,
           move_type_indices, float_features, pokemon_table, move_table,
           ability_table, item_table, type_table, W1, b1, W2, b2):
  f32 = jnp.float32
  mv_tab = jnp.pad(
      jnp.concatenate([move_table, jnp.zeros((1, 24), f32)], axis=0),
      ((0, 0), (0, 8)))
  ty_tab = jnp.concatenate([type_table, jnp.zeros((1, 16), f32)], axis=0)
  w1p = jnp.concatenate(
      [W1[:72], jnp.zeros((8, 256), f32), W1[72:], jnp.zeros((1, 256), f32)],
      axis=0)

  (se, m0, m1, m2, m3, ae, ie, t1, t2, mt0, mt1, mt2, mt3, rm, rt) = (
      _sc_gather(
          species_idx,
          move_indices[:, 0], move_indices[:, 1],
          move_indices[:, 2], move_indices[:, 3],
          ability_idx, item_idx,
          type_indices[:, 0], type_indices[:, 1],
          move_type_indices[:, 0], move_type_indices[:, 1],
          move_type_indices[:, 2], move_type_indices[:, 3],
          pokemon_table, mv_tab, ability_table, item_table, ty_tab))

  return _tc_mlp(se, m0, m1, m2, m3, ae, ie, t1, t2, mt0, mt1, mt2, mt3,
                 rm.reshape(_B, 1), rt.reshape(_B, 1), float_features,
                 w1p, b1.reshape(1, 256), W2, b2.reshape(1, 128))


# speedup vs baseline: 3.0190x; 1.1183x over previous
import jax
import jax.numpy as jnp
from jax import lax
from jax.experimental import pallas as pl
from jax.experimental.pallas import tpu as pltpu
from jax.experimental.pallas import tpu_sc as plsc

_B = 16384
_NW = 32
_RW = _B // _NW
_CHUNK = 128
_NCHUNK = _RW // _CHUNK

_COLS = (
    ("se", 0, 48), ("m0", 48, 32), ("m1", 80, 32), ("m2", 112, 32),
    ("m3", 144, 32), ("ae", 176, 16), ("ie", 192, 16), ("t1", 208, 16),
    ("t2", 224, 16), ("mt0", 240, 16), ("mt1", 256, 16), ("mt2", 272, 16),
    ("mt3", 288, 16),
)
_XW = 304


def _sc_gather_kernel(
    sp_h, mv0_h, mv1_h, mv2_h, mv3_h, ab_h, it_h,
    t1_h, t2_h, mt0_h, mt1_h, mt2_h, mt3_h,
    pok_tab, mv_tab, ab_tab, it_tab, ty_tab,
    x_o, rm_o, rt_o,
    sp_i, mv0_i, mv1_i, mv2_i, mv3_i, ab_i, it_i, t1_i, t2_i,
    mt0_i, mt1_i, mt2_i, mt3_i,
    se_a, m0_a, m1_a, m2_a, m3_a, ae_a, ie_a, t1_a, t2_a,
    mt0_a, mt1_a, mt2_a, mt3_a,
    se_b, m0_b, m1_b, m2_b, m3_b, ae_b, ie_b, t1_b, t2_b,
    mt0_b, mt1_b, mt2_b, mt3_b,
    rm_v, rt_v, isem, gsem, wsem,
):
  wid = lax.axis_index("c") * 16 + lax.axis_index("s")
  base = wid * _RW
  bsl = pl.ds(base, _RW)

  idx_bufs = (sp_i, mv0_i, mv1_i, mv2_i, mv3_i, ab_i, it_i, t1_i, t2_i,
              mt0_i, mt1_i, mt2_i, mt3_i)
  idx_hbm = (sp_h, mv0_h, mv1_h, mv2_h, mv3_h, ab_h, it_h, t1_h, t2_h,
             mt0_h, mt1_h, mt2_h, mt3_h)
  tabs = (pok_tab, mv_tab, mv_tab, mv_tab, mv_tab, ab_tab, it_tab,
          ty_tab, ty_tab, ty_tab, ty_tab, ty_tab, ty_tab)
  row_bufs = (
      (se_a, m0_a, m1_a, m2_a, m3_a, ae_a, ie_a, t1_a, t2_a,
       mt0_a, mt1_a, mt2_a, mt3_a),
      (se_b, m0_b, m1_b, m2_b, m3_b, ae_b, ie_b, t1_b, t2_b,
       mt0_b, mt1_b, mt2_b, mt3_b),
  )

  ih = [pltpu.async_copy(h.at[bsl], buf, isem)
        for h, buf in zip(idx_hbm, idx_bufs)]
  for h in ih:
    h.wait()

  mv_is = (mv0_i, mv1_i, mv2_i, mv3_i)
  mt_is = (mt0_i, mt1_i, mt2_i, mt3_i)
  for i in range(_RW // 16):
    s2 = pl.ds(i * 16, 16)
    cm = jnp.zeros((16,), jnp.float32)
    ct = jnp.zeros((16,), jnp.float32)
    for j in range(4):
      v = mv_is[j][s2]
      nz = v != 0
      cm = cm + jnp.where(nz, 1.0, 0.0)
      mv_is[j][s2] = jnp.where(nz, v, 920)
      w = mt_is[j][s2]
      nzt = w != 0
      ct = ct + jnp.where(nzt, 1.0, 0.0)
      mt_is[j][s2] = jnp.where(nzt, w, 19)
    rm_v[s2] = 1.0 / jnp.maximum(cm, 1.0)
    rt_v[s2] = 1.0 / jnp.maximum(ct, 1.0)

  wh = [None, None]
  for c in range(_NCHUNK):
    bset = row_bufs[c % 2]
    csl = pl.ds(c * _CHUNK, _CHUNK)
    osl = pl.ds(base + c * _CHUNK, _CHUNK)
    if wh[c % 2] is not None:
      for h in wh[c % 2]:
        h.wait()
    gh = [pltpu.async_copy(tab.at[ibuf.at[csl]], rbuf, gsem)
          for tab, ibuf, rbuf in zip(tabs, idx_bufs, bset)]
    for h in gh:
      h.wait()
    wh[c % 2] = [
        pltpu.async_copy(rbuf, x_o.at[osl, pl.ds(col, w)], wsem)
        for rbuf, (_, col, w) in zip(bset, _COLS)
    ]
  for hs in wh:
    if hs is not None:
      for h in hs:
        h.wait()
  pltpu.sync_copy(rm_v, rm_o.at[bsl])
  pltpu.sync_copy(rt_v, rt_o.at[bsl])


def _make_sc_gather():
  f32 = jnp.float32
  i32 = jnp.int32
  out_type = [
      jax.ShapeDtypeStruct((_B, _XW), f32),
      jax.ShapeDtypeStruct((_B,), f32),
      jax.ShapeDtypeStruct((_B,), f32),
  ]
  widths = [w for (_, _, w) in _COLS]
  rowset = [pltpu.VMEM((_CHUNK, w), f32) for w in widths]
  scratch = [
      *[pltpu.VMEM((_RW,), i32) for _ in range(13)],
      *rowset, *rowset,
      pltpu.VMEM((_RW,), f32),
      pltpu.VMEM((_RW,), f32),
      pltpu.SemaphoreType.DMA,
      pltpu.SemaphoreType.DMA,
      pltpu.SemaphoreType.DMA,
  ]
  mesh = plsc.VectorSubcoreMesh(core_axis_name="c", subcore_axis_name="s")
  return pl.kernel(
      _sc_gather_kernel, out_type=out_type, mesh=mesh,
      scratch_types=scratch,
      compiler_params=pltpu.CompilerParams(use_tc_tiling_on_sc=False))


_sc_gather = _make_sc_gather()

_BS = 512


def _tc_mlp_kernel(x, rm, rt, ff, w1, b1, w2, b2, out):
  xv = x[...]
  msum = (xv[:, 48:80] + xv[:, 80:112] + xv[:, 112:144] + xv[:, 144:176]
          ) * rm[...]
  mtsum = (xv[:, 240:256] + xv[:, 256:272] + xv[:, 272:288] + xv[:, 288:304]
           ) * rt[...]
  xc = jnp.concatenate(
      [xv[:, 0:48], msum, xv[:, 176:240], mtsum, ff[...],
       jnp.zeros((_BS, 1), jnp.float32)], axis=-1)
  h = jnp.maximum(
      jnp.dot(xc, w1[...], preferred_element_type=jnp.float32) + b1[...], 0.0)
  out[...] = jnp.maximum(
      jnp.dot(h, w2[...], preferred_element_type=jnp.float32) + b2[...], 0.0)


def _make_tc_mlp():
  def bspec(cols):
    return pl.BlockSpec((_BS, cols), lambda i: (i, 0))
  in_specs = [
      bspec(_XW),
      bspec(1), bspec(1),
      bspec(31),
      pl.BlockSpec((192, 256), lambda i: (0, 0)),
      pl.BlockSpec((1, 256), lambda i: (0, 0)),
      pl.BlockSpec((256, 128), lambda i: (0, 0)),
      pl.BlockSpec((1, 128), lambda i: (0, 0)),
  ]
  return pl.pallas_call(
      _tc_mlp_kernel,
      grid=(_B // _BS,),
      in_specs=in_specs,
      out_specs=pl.BlockSpec((_BS, 128), lambda i: (i, 0)),
      out_shape=jax.ShapeDtypeStruct((_B, 128), jnp.float32),
  )


_tc_mlp = _make_tc_mlp()


def kernel(species_idx, move_indices, ability_idx, item_idx, type_indices,
           move_type_indices, float_features, pokemon_table, move_table,
           ability_table, item_table, type_table, W1, b1, W2, b2):
  f32 = jnp.float32
  mv_tab = jnp.pad(
      jnp.concatenate([move_table, jnp.zeros((1, 24), f32)], axis=0),
      ((0, 0), (0, 8)))
  ty_tab = jnp.concatenate([type_table, jnp.zeros((1, 16), f32)], axis=0)
  w1p = jnp.concatenate(
      [W1[:72], jnp.zeros((8, 256), f32), W1[72:], jnp.zeros((1, 256), f32)],
      axis=0)

  x, rm, rt = _sc_gather(
      species_idx,
      move_indices[:, 0], move_indices[:, 1],
      move_indices[:, 2], move_indices[:, 3],
      ability_idx, item_idx,
      type_indices[:, 0], type_indices[:, 1],
      move_type_indices[:, 0], move_type_indices[:, 1],
      move_type_indices[:, 2], move_type_indices[:, 3],
      pokemon_table, mv_tab, ability_table, item_table, ty_tab)

  return _tc_mlp(x, rm.reshape(_B, 1), rt.reshape(_B, 1), float_features,
                 w1p, b1.reshape(1, 256), W2, b2.reshape(1, 128))


# speedup vs baseline: 6.6646x; 2.2075x over previous
import jax
import jax.numpy as jnp
from jax import lax
from jax.experimental import pallas as pl
from jax.experimental.pallas import tpu as pltpu
from jax.experimental.pallas import tpu_sc as plsc

_B = 16384
_NW = 32
_RW = _B // _NW
_G = 32
_NG = _RW // _G
_XW = 160


def _sc_gather_kernel(
    sp_h, mv0_h, mv1_h, mv2_h, mv3_h, ab_h, it_h,
    t1_h, t2_h, mt0_h, mt1_h, mt2_h, mt3_h,
    pok_h, mv_h, ab_tab_h, it_tab_h, ty_h,
    x_o, rm_o, rt_o,
    pok_v, mv_v, ab_v, it_v, ty_v,
    sp_i, mv0_i, mv1_i, mv2_i, mv3_i, ab_i, it_i, t1_i, t2_i,
    mt0_i, mt1_i, mt2_i, mt3_i,
    xball, rm_v, rt_v, isem, wsem,
):
  wid = lax.axis_index("c") * 16 + lax.axis_index("s")
  base = wid * _RW
  bsl = pl.ds(base, _RW)

  idx_bufs = (sp_i, mv0_i, mv1_i, mv2_i, mv3_i, ab_i, it_i, t1_i, t2_i,
              mt0_i, mt1_i, mt2_i, mt3_i)
  idx_hbm = (sp_h, mv0_h, mv1_h, mv2_h, mv3_h, ab_h, it_h, t1_h, t2_h,
             mt0_h, mt1_h, mt2_h, mt3_h)

  ih = [pltpu.async_copy(h, v, isem) for h, v in
        ((pok_h, pok_v), (mv_h, mv_v), (ab_tab_h, ab_v), (it_tab_h, it_v),
         (ty_h, ty_v))]
  ih += [pltpu.async_copy(h.at[bsl], buf, isem)
         for h, buf in zip(idx_hbm, idx_bufs)]
  for h in ih:
    h.wait()

  mv_is = (mv0_i, mv1_i, mv2_i, mv3_i)
  mt_is = (mt0_i, mt1_i, mt2_i, mt3_i)

  def remap_body(i, _):
    s2 = pl.ds(i * 16, 16)
    cm = jnp.zeros((16,), jnp.float32)
    ct = jnp.zeros((16,), jnp.float32)
    for j in range(4):
      v = mv_is[j][s2]
      nz = v != 0
      cm = cm + jnp.where(nz, 1.0, 0.0)
      mv_is[j][s2] = jnp.where(nz, v, 920)
      w = mt_is[j][s2]
      nzt = w != 0
      ct = ct + jnp.where(nzt, 1.0, 0.0)
      mt_is[j][s2] = jnp.where(nzt, w, 19)
    rm_v[s2] = 1.0 / jnp.maximum(cm, 1.0)
    rt_v[s2] = 1.0 / jnp.maximum(ct, 1.0)
    return 0

  lax.fori_loop(0, _RW // 16, remap_body, 0)

  gsz = _G * _XW

  def grp_body(g, _):
    obase = (g % 2) * gsz

    @pl.when(g >= 2)
    def _reclaim():
      pltpu.make_async_copy(
          xball.at[pl.ds(0, gsz)],
          x_o.at[pl.ds(base * _XW, gsz)], wsem).wait()

    for sub in range(_G // 16):
      ssl = pl.ds((g * (_G // 16) + sub) * 16, 16)
      sib = sp_i[ssl] * 48
      i0b = mv0_i[ssl] * 32
      i1b = mv1_i[ssl] * 32
      i2b = mv2_i[ssl] * 32
      i3b = mv3_i[ssl] * 32
      abb = ab_i[ssl] * 16
      itb = it_i[ssl] * 16
      t1b = t1_i[ssl] * 16
      t2b = t2_i[ssl] * 16
      u0b = mt0_i[ssl] * 16
      u1b = mt1_i[ssl] * 16
      u2b = mt2_i[ssl] * 16
      u3b = mt3_i[ssl] * 16
      for j in range(16):
        o = obase + (sub * 16 + j) * _XW
        si = sib[j]
        for c in range(3):
          xball[pl.ds(o + c * 16, 16)] = pok_v[pl.ds(si + c * 16, 16)]
        i0 = i0b[j]
        i1 = i1b[j]
        i2 = i2b[j]
        i3 = i3b[j]
        for c in range(2):
          acc = (mv_v[pl.ds(i0 + c * 16, 16)] + mv_v[pl.ds(i1 + c * 16, 16)]
                 + mv_v[pl.ds(i2 + c * 16, 16)]
                 + mv_v[pl.ds(i3 + c * 16, 16)])
          xball[pl.ds(o + 48 + c * 16, 16)] = acc
        xball[pl.ds(o + 80, 16)] = ab_v[pl.ds(abb[j], 16)]
        xball[pl.ds(o + 96, 16)] = it_v[pl.ds(itb[j], 16)]
        xball[pl.ds(o + 112, 16)] = ty_v[pl.ds(t1b[j], 16)]
        xball[pl.ds(o + 128, 16)] = ty_v[pl.ds(t2b[j], 16)]
        tacc = (ty_v[pl.ds(u0b[j], 16)] + ty_v[pl.ds(u1b[j], 16)]
                + ty_v[pl.ds(u2b[j], 16)] + ty_v[pl.ds(u3b[j], 16)])
        xball[pl.ds(o + 144, 16)] = tacc
    pltpu.async_copy(
        xball.at[pl.ds(obase, gsz)],
        x_o.at[pl.ds((base + g * _G) * _XW, gsz)], wsem)
    return 0

  lax.fori_loop(0, _NG, grp_body, 0)
  for _ in range(2):
    pltpu.make_async_copy(
        xball.at[pl.ds(0, gsz)],
        x_o.at[pl.ds(base * _XW, gsz)], wsem).wait()
  pltpu.sync_copy(rm_v, rm_o.at[bsl])
  pltpu.sync_copy(rt_v, rt_o.at[bsl])


def _make_sc_gather():
  f32 = jnp.float32
  i32 = jnp.int32
  out_type = [
      jax.ShapeDtypeStruct((_B * _XW,), f32),
      jax.ShapeDtypeStruct((_B,), f32),
      jax.ShapeDtypeStruct((_B,), f32),
  ]
  scratch = [
      pltpu.VMEM((1025 * 48,), f32),
      pltpu.VMEM((921 * 32,), f32),
      pltpu.VMEM((310 * 16,), f32),
      pltpu.VMEM((1200 * 16,), f32),
      pltpu.VMEM((20 * 16,), f32),
      *[pltpu.VMEM((_RW,), i32) for _ in range(13)],
      pltpu.VMEM((2 * _G * _XW,), f32),
      pltpu.VMEM((_RW,), f32),
      pltpu.VMEM((_RW,), f32),
      pltpu.SemaphoreType.DMA,
      pltpu.SemaphoreType.DMA,
  ]
  mesh = plsc.VectorSubcoreMesh(core_axis_name="c", subcore_axis_name="s")
  return pl.kernel(
      _sc_gather_kernel, out_type=out_type, mesh=mesh,
      scratch_types=scratch,
      compiler_params=pltpu.CompilerParams(use_tc_tiling_on_sc=False))


_sc_gather = _make_sc_gather()

_BS = 512


def _tc_mlp_kernel(x, rm, rt, ff, w1, b1, w2, b2, out):
  xv = x[...]
  xc = jnp.concatenate(
      [xv[:, 0:48], xv[:, 48:80] * rm[...], xv[:, 80:144],
       xv[:, 144:160] * rt[...], ff[...],
       jnp.zeros((_BS, 1), jnp.float32)], axis=-1)
  h = jnp.maximum(
      jnp.dot(xc, w1[...], preferred_element_type=jnp.float32) + b1[...], 0.0)
  out[...] = jnp.maximum(
      jnp.dot(h, w2[...], preferred_element_type=jnp.float32) + b2[...], 0.0)


def _make_tc_mlp():
  def bspec(cols):
    return pl.BlockSpec((_BS, cols), lambda i: (i, 0))
  in_specs = [
      bspec(_XW),
      bspec(1), bspec(1),
      bspec(31),
      pl.BlockSpec((192, 256), lambda i: (0, 0)),
      pl.BlockSpec((1, 256), lambda i: (0, 0)),
      pl.BlockSpec((256, 128), lambda i: (0, 0)),
      pl.BlockSpec((1, 128), lambda i: (0, 0)),
  ]
  return pl.pallas_call(
      _tc_mlp_kernel,
      grid=(_B // _BS,),
      in_specs=in_specs,
      out_specs=pl.BlockSpec((_BS, 128), lambda i: (i, 0)),
      out_shape=jax.ShapeDtypeStruct((_B, 128), jnp.float32),
  )


_tc_mlp = _make_tc_mlp()


def kernel(species_idx, move_indices, ability_idx, item_idx, type_indices,
           move_type_indices, float_features, pokemon_table, move_table,
           ability_table, item_table, type_table, W1, b1, W2, b2):
  f32 = jnp.float32
  mv_tab = jnp.pad(
      jnp.concatenate([move_table, jnp.zeros((1, 24), f32)], axis=0),
      ((0, 0), (0, 8)))
  ty_tab = jnp.concatenate([type_table, jnp.zeros((1, 16), f32)], axis=0)
  w1p = jnp.concatenate(
      [W1[:72], jnp.zeros((8, 256), f32), W1[72:], jnp.zeros((1, 256), f32)],
      axis=0)

  x, rm, rt = _sc_gather(
      species_idx,
      move_indices[:, 0], move_indices[:, 1],
      move_indices[:, 2], move_indices[:, 3],
      ability_idx, item_idx,
      type_indices[:, 0], type_indices[:, 1],
      move_type_indices[:, 0], move_type_indices[:, 1],
      move_type_indices[:, 2], move_type_indices[:, 3],
      pokemon_table.reshape(-1), mv_tab.reshape(-1),
      ability_table.reshape(-1), item_table.reshape(-1), ty_tab.reshape(-1))

  return _tc_mlp(x.reshape(_B, _XW), rm.reshape(_B, 1), rt.reshape(_B, 1),
                 float_features, w1p, b1.reshape(1, 256), W2,
                 b2.reshape(1, 128))


# speedup vs baseline: 7.3618x; 1.1046x over previous
import jax
import jax.numpy as jnp
from jax import lax
from jax.experimental import pallas as pl
from jax.experimental.pallas import tpu as pltpu
from jax.experimental.pallas import tpu_sc as plsc

_B = 16384
_NW = 32
_RW = _B // _NW
_G = 32
_NG = _RW // _G
_XW = 128


def _sc_gather_kernel(
    sp_h, mv0_h, mv1_h, mv2_h, mv3_h, ab_h, it_h, t1_h,
    pok_h, mv_h, ab_tab_h, it_tab_h, ty_h,
    x_o,
    pok_v, mv_v, ab_v, it_v, ty_v,
    sp_i, mv0_i, mv1_i, mv2_i, mv3_i, ab_i, it_i, t1_i,
    xball, isem, wsem,
):
  wid = lax.axis_index("c") * 16 + lax.axis_index("s")
  base = wid * _RW
  bsl = pl.ds(base, _RW)

  idx_bufs = (sp_i, mv0_i, mv1_i, mv2_i, mv3_i, ab_i, it_i, t1_i)
  idx_hbm = (sp_h, mv0_h, mv1_h, mv2_h, mv3_h, ab_h, it_h, t1_h)

  ih = [pltpu.async_copy(h, v, isem) for h, v in
        ((pok_h, pok_v), (mv_h, mv_v), (ab_tab_h, ab_v), (it_tab_h, it_v),
         (ty_h, ty_v))]
  ih += [pltpu.async_copy(h.at[bsl], buf, isem)
         for h, buf in zip(idx_hbm, idx_bufs)]
  for h in ih:
    h.wait()

  mv_is = (mv0_i, mv1_i, mv2_i, mv3_i)

  def remap_body(i, _):
    s2 = pl.ds(i * 16, 16)
    for j in range(4):
      v = mv_is[j][s2]
      mv_is[j][s2] = jnp.where(v != 0, v, 920)
    return 0

  lax.fori_loop(0, _RW // 16, remap_body, 0)

  gsz = _G * _XW

  def grp_body(g, _):
    obase = (g % 2) * gsz

    @pl.when(g >= 2)
    def _reclaim():
      pltpu.make_async_copy(
          xball.at[pl.ds(0, gsz)],
          x_o.at[pl.ds(base * _XW, gsz)], wsem).wait()

    for sub in range(_G // 16):
      ssl = pl.ds((g * (_G // 16) + sub) * 16, 16)
      sib = sp_i[ssl] * 48
      i0b = mv0_i[ssl] * 32
      i1b = mv1_i[ssl] * 32
      i2b = mv2_i[ssl] * 32
      i3b = mv3_i[ssl] * 32
      abb = ab_i[ssl] * 16
      itb = it_i[ssl] * 16
      t1b = t1_i[ssl] * 16
      for j in range(16):
        o = obase + (sub * 16 + j) * _XW
        si = sib[j]
        for c in range(3):
          xball[pl.ds(o + c * 16, 16)] = pok_v[pl.ds(si + c * 16, 16)]
        i0 = i0b[j]
        i1 = i1b[j]
        i2 = i2b[j]
        i3 = i3b[j]
        for c in range(2):
          acc = (mv_v[pl.ds(i0 + c * 16, 16)] + mv_v[pl.ds(i1 + c * 16, 16)]
                 + mv_v[pl.ds(i2 + c * 16, 16)]
                 + mv_v[pl.ds(i3 + c * 16, 16)])
          xball[pl.ds(o + 48 + c * 16, 16)] = acc
        xball[pl.ds(o + 80, 16)] = ab_v[pl.ds(abb[j], 16)]
        xball[pl.ds(o + 96, 16)] = it_v[pl.ds(itb[j], 16)]
        xball[pl.ds(o + 112, 16)] = ty_v[pl.ds(t1b[j], 16)]
    pltpu.async_copy(
        xball.at[pl.ds(obase, gsz)],
        x_o.at[pl.ds((base + g * _G) * _XW, gsz)], wsem)
    return 0

  lax.fori_loop(0, _NG, grp_body, 0)
  for _ in range(2):
    pltpu.make_async_copy(
        xball.at[pl.ds(0, gsz)],
        x_o.at[pl.ds(base * _XW, gsz)], wsem).wait()


def _make_sc_gather():
  f32 = jnp.float32
  i32 = jnp.int32
  out_type = [
      jax.ShapeDtypeStruct((_B * _XW,), f32),
  ]
  scratch = [
      pltpu.VMEM((1025 * 48,), f32),
      pltpu.VMEM((921 * 32,), f32),
      pltpu.VMEM((310 * 16,), f32),
      pltpu.VMEM((1200 * 16,), f32),
      pltpu.VMEM((19 * 16,), f32),
      *[pltpu.VMEM((_RW,), i32) for _ in range(8)],
      pltpu.VMEM((2 * _G * _XW,), f32),
      pltpu.SemaphoreType.DMA,
      pltpu.SemaphoreType.DMA,
  ]
  mesh = plsc.VectorSubcoreMesh(core_axis_name="c", subcore_axis_name="s")
  return pl.kernel(
      _sc_gather_kernel, out_type=out_type, mesh=mesh,
      scratch_types=scratch,
      compiler_params=pltpu.CompilerParams(use_tc_tiling_on_sc=False))


_sc_gather = _make_sc_gather()

_BS = 512


def _tc_mlp_kernel(x, mvi, tyi, mti, ff, tytab, w1, b1, w2, b2, out):
  f32 = jnp.float32
  xv = x[...]

  mv = mvi[...]
  rm = 1.0 / jnp.maximum(jnp.sum((mv != 0).astype(f32), axis=1,
                                 keepdims=True), 1.0)
  msum = xv[:, 48:80] * rm

  cols = lax.broadcasted_iota(jnp.int32, (_BS, 32), 1)
  t2 = tyi[...][:, 1:2]
  t2e = jnp.dot((cols == t2).astype(f32), tytab[...],
                preferred_element_type=f32)
  mt = mti[...]
  ohsum = jnp.zeros((_BS, 32), f32)
  for j in range(4):
    c = mt[:, j:j + 1]
    ohsum = ohsum + ((cols == c) & (c != 0)).astype(f32)
  rt = 1.0 / jnp.maximum(jnp.sum((mt != 0).astype(f32), axis=1,
                                 keepdims=True), 1.0)
  mte = jnp.dot(ohsum, tytab[...], preferred_element_type=f32) * rt

  xc = jnp.concatenate(
      [xv[:, 0:48], msum, xv[:, 80:128], t2e, mte, ff[...],
       jnp.zeros((_BS, 1), f32)], axis=-1)
  h = jnp.maximum(
      jnp.dot(xc, w1[...], preferred_element_type=f32) + b1[...], 0.0)
  out[...] = jnp.maximum(
      jnp.dot(h, w2[...], preferred_element_type=f32) + b2[...], 0.0)


def _make_tc_mlp():
  def bspec(cols):
    return pl.BlockSpec((_BS, cols), lambda i: (i, 0))
  in_specs = [
      bspec(_XW),
      bspec(4),
      bspec(2),
      bspec(4),
      bspec(31),
      pl.BlockSpec((32, 16), lambda i: (0, 0)),
      pl.BlockSpec((192, 256), lambda i: (0, 0)),
      pl.BlockSpec((1, 256), lambda i: (0, 0)),
      pl.BlockSpec((256, 128), lambda i: (0, 0)),
      pl.BlockSpec((1, 128), lambda i: (0, 0)),
  ]
  return pl.pallas_call(
      _tc_mlp_kernel,
      grid=(_B // _BS,),
      in_specs=in_specs,
      out_specs=pl.BlockSpec((_BS, 128), lambda i: (i, 0)),
      out_shape=jax.ShapeDtypeStruct((_B, 128), jnp.float32),
  )


_tc_mlp = _make_tc_mlp()


def kernel(species_idx, move_indices, ability_idx, item_idx, type_indices,
           move_type_indices, float_features, pokemon_table, move_table,
           ability_table, item_table, type_table, W1, b1, W2, b2):
  f32 = jnp.float32
  mv_tab = jnp.pad(
      jnp.concatenate([move_table, jnp.zeros((1, 24), f32)], axis=0),
      ((0, 0), (0, 8)))
  ty_pad = jnp.pad(type_table, ((0, 13), (0, 0)))
  w1p = jnp.concatenate(
      [W1[:72], jnp.zeros((8, 256), f32), W1[72:], jnp.zeros((1, 256), f32)],
      axis=0)

  (x,) = _sc_gather(
      species_idx,
      move_indices[:, 0], move_indices[:, 1],
      move_indices[:, 2], move_indices[:, 3],
      ability_idx, item_idx, type_indices[:, 0],
      pokemon_table.reshape(-1), mv_tab.reshape(-1),
      ability_table.reshape(-1), item_table.reshape(-1),
      type_table.reshape(-1))

  return _tc_mlp(x.reshape(_B, _XW), move_indices, type_indices,
                 move_type_indices, float_features, ty_pad, w1p,
                 b1.reshape(1, 256), W2, b2.reshape(1, 128))


# speedup vs baseline: 8.2377x; 1.1190x over previous
import jax
import jax.numpy as jnp
from jax import lax
from jax.experimental import pallas as pl
from jax.experimental.pallas import tpu as pltpu
from jax.experimental.pallas import tpu_sc as plsc

_B = 16384
_NW = 32
_RW = _B // _NW
_G = 32
_NG = _RW // _G
_XW = 128


def _sc_gather_kernel(
    sp_h, mv0_h, mv1_h, mv2_h, mv3_h, ab_h, it_h, t1_h,
    pok_h, mv_h, ab_tab_h, it_tab_h, ty_h,
    x_o,
    pok_v, mv_v, ab_v, it_v, ty_v,
    sp_i, mv0_i, mv1_i, mv2_i, mv3_i, ab_i, it_i, t1_i,
    xball, isem, wsem,
):
  wid = lax.axis_index("c") * 16 + lax.axis_index("s")
  base = wid * _RW
  bsl = pl.ds(base, _RW)

  idx_bufs = (sp_i, mv0_i, mv1_i, mv2_i, mv3_i, ab_i, it_i, t1_i)
  idx_hbm = (sp_h, mv0_h, mv1_h, mv2_h, mv3_h, ab_h, it_h, t1_h)

  ih = [pltpu.async_copy(h, v, isem) for h, v in
        ((pok_h, pok_v), (mv_h, mv_v), (ab_tab_h, ab_v), (it_tab_h, it_v),
         (ty_h, ty_v))]
  ih += [pltpu.async_copy(h.at[bsl], buf, isem)
         for h, buf in zip(idx_hbm, idx_bufs)]
  for h in ih:
    h.wait()

  mv_is = (mv0_i, mv1_i, mv2_i, mv3_i)

  def remap_body(i, _):
    s2 = pl.ds(i * 16, 16)
    for j in range(4):
      v = mv_is[j][s2]
      mv_is[j][s2] = jnp.where(v != 0, v, 920)
    return 0

  lax.fori_loop(0, _RW // 16, remap_body, 0)

  gsz = _G * _XW

  def grp_body(g, _):
    obase = (g % 2) * gsz

    @pl.when(g >= 2)
    def _reclaim():
      pltpu.make_async_copy(
          xball.at[pl.ds(0, gsz)],
          x_o.at[pl.ds(base * _XW, gsz)], wsem).wait()

    for sub in range(_G // 16):
      ssl = pl.ds((g * (_G // 16) + sub) * 16, 16)
      sib = sp_i[ssl] * 48
      i0b = mv0_i[ssl] * 32
      i1b = mv1_i[ssl] * 32
      i2b = mv2_i[ssl] * 32
      i3b = mv3_i[ssl] * 32
      abb = ab_i[ssl] * 16
      itb = it_i[ssl] * 16
      t1b = t1_i[ssl] * 16
      for j in range(16):
        o = obase + (sub * 16 + j) * _XW
        si = sib[j]
        for c in range(3):
          xball[pl.ds(o + c * 16, 16)] = pok_v[pl.ds(si + c * 16, 16)]
        i0 = i0b[j]
        i1 = i1b[j]
        i2 = i2b[j]
        i3 = i3b[j]
        for c in range(2):
          acc = (mv_v[pl.ds(i0 + c * 16, 16)] + mv_v[pl.ds(i1 + c * 16, 16)]
                 + mv_v[pl.ds(i2 + c * 16, 16)]
                 + mv_v[pl.ds(i3 + c * 16, 16)])
          xball[pl.ds(o + 48 + c * 16, 16)] = acc
        xball[pl.ds(o + 80, 16)] = ab_v[pl.ds(abb[j], 16)]
        xball[pl.ds(o + 96, 16)] = it_v[pl.ds(itb[j], 16)]
        xball[pl.ds(o + 112, 16)] = ty_v[pl.ds(t1b[j], 16)]
    pltpu.async_copy(
        xball.at[pl.ds(obase, gsz)],
        x_o.at[pl.ds((base + g * _G) * _XW, gsz)], wsem)
    return 0

  lax.fori_loop(0, _NG, grp_body, 0)
  for _ in range(2):
    pltpu.make_async_copy(
        xball.at[pl.ds(0, gsz)],
        x_o.at[pl.ds(base * _XW, gsz)], wsem).wait()


def _make_sc_gather():
  f32 = jnp.float32
  i32 = jnp.int32
  out_type = [
      jax.ShapeDtypeStruct((_B * _XW,), f32),
  ]
  scratch = [
      pltpu.VMEM((1025 * 48,), f32),
      pltpu.VMEM((921 * 32,), f32),
      pltpu.VMEM((310 * 16,), f32),
      pltpu.VMEM((1200 * 16,), f32),
      pltpu.VMEM((19 * 16,), f32),
      *[pltpu.VMEM((_RW,), i32) for _ in range(8)],
      pltpu.VMEM((2 * _G * _XW,), f32),
      pltpu.SemaphoreType.DMA,
      pltpu.SemaphoreType.DMA,
  ]
  mesh = plsc.VectorSubcoreMesh(core_axis_name="c", subcore_axis_name="s")
  return pl.kernel(
      _sc_gather_kernel, out_type=out_type, mesh=mesh,
      scratch_types=scratch,
      compiler_params=pltpu.CompilerParams(use_tc_tiling_on_sc=False))


_sc_gather = _make_sc_gather()

_BS = 1024


def _tc_mlp_kernel(x, mvi, tyi, mti, ff, tytab, w1a, w1t2, w1mt, w1ff,
                   b1, w2, b2, out):
  f32 = jnp.float32
  xv = x[...]

  mv = mvi[...]
  nz = (mv != 0).astype(f32)
  cnt = nz[:, 0:1] + nz[:, 1:2] + nz[:, 2:3] + nz[:, 3:4]
  rm = 1.0 / jnp.maximum(cnt, 1.0)
  cols128 = lax.broadcasted_iota(jnp.int32, (_BS, _XW), 1)
  xs = xv * jnp.where((cols128 >= 48) & (cols128 < 80), rm, 1.0)

  cols = lax.broadcasted_iota(jnp.int32, (_BS, 32), 1)
  t2 = tyi[...][:, 1:2]
  oh2 = (cols == t2).astype(f32)
  mt = mti[...]
  mtnz = (mt != 0)
  ohsum = jnp.zeros((_BS, 32), f32)
  for j in range(4):
    c = mt[:, j:j + 1]
    ohsum = ohsum + ((cols == c) & (c != 0)).astype(f32)
  ctf = mtnz.astype(f32)
  ct = ctf[:, 0:1] + ctf[:, 1:2] + ctf[:, 2:3] + ctf[:, 3:4]
  ohs = ohsum * (1.0 / jnp.maximum(ct, 1.0))

  p2 = jnp.dot(tytab[...], w1t2[...], preferred_element_type=f32)
  pt = jnp.dot(tytab[...], w1mt[...], preferred_element_type=f32)
  h = (jnp.dot(xs, w1a[...], preferred_element_type=f32)
       + jnp.dot(oh2, p2, preferred_element_type=f32)
       + jnp.dot(ohs, pt, preferred_element_type=f32)
       + jnp.dot(ff[...], w1ff[...], preferred_element_type=f32)
       + b1[...])
  h = jnp.maximum(h, 0.0)
  out[...] = jnp.maximum(
      jnp.dot(h, w2[...], preferred_element_type=f32) + b2[...], 0.0)


def _make_tc_mlp():
  def bspec(cols):
    return pl.BlockSpec((_BS, cols), lambda i: (i, 0))
  in_specs = [
      bspec(_XW),
      bspec(4),
      bspec(2),
      bspec(4),
      bspec(31),
      pl.BlockSpec((32, 16), lambda i: (0, 0)),
      pl.BlockSpec((128, 256), lambda i: (0, 0)),
      pl.BlockSpec((16, 256), lambda i: (0, 0)),
      pl.BlockSpec((16, 256), lambda i: (0, 0)),
      pl.BlockSpec((31, 256), lambda i: (0, 0)),
      pl.BlockSpec((1, 256), lambda i: (0, 0)),
      pl.BlockSpec((256, 128), lambda i: (0, 0)),
      pl.BlockSpec((1, 128), lambda i: (0, 0)),
  ]
  return pl.pallas_call(
      _tc_mlp_kernel,
      grid=(_B // _BS,),
      in_specs=in_specs,
      out_specs=pl.BlockSpec((_BS, 128), lambda i: (i, 0)),
      out_shape=jax.ShapeDtypeStruct((_B, 128), jnp.float32),
  )


_tc_mlp = _make_tc_mlp()


def kernel(species_idx, move_indices, ability_idx, item_idx, type_indices,
           move_type_indices, float_features, pokemon_table, move_table,
           ability_table, item_table, type_table, W1, b1, W2, b2):
  f32 = jnp.float32
  mv_tab = jnp.pad(
      jnp.concatenate([move_table, jnp.zeros((1, 24), f32)], axis=0),
      ((0, 0), (0, 8)))
  ty_pad = jnp.pad(type_table, ((0, 13), (0, 0)))
  w1a = jnp.concatenate([W1[:72], jnp.zeros((8, 256), f32), W1[72:120]],
                        axis=0)
  w1t2 = W1[120:136]
  w1mt = W1[136:152]
  w1ff = W1[152:183]

  (x,) = _sc_gather(
      species_idx,
      move_indices[:, 0], move_indices[:, 1],
      move_indices[:, 2], move_indices[:, 3],
      ability_idx, item_idx, type_indices[:, 0],
      pokemon_table.reshape(-1), mv_tab.reshape(-1),
      ability_table.reshape(-1), item_table.reshape(-1),
      type_table.reshape(-1))

  return _tc_mlp(x.reshape(_B, _XW), move_indices, type_indices,
                 move_type_indices, float_features, ty_pad, w1a, w1t2,
                 w1mt, w1ff, b1.reshape(1, 256), W2, b2.reshape(1, 128))
